# Initial kernel scaffold; baseline (speedup 1.0000x reference)
import jax
import jax.numpy as jnp
from jax.experimental import pallas as pl


def kernel(node_features, edge_index, edge_attr, W1, as1, ad1, We1, ae1, b1, W2, as2, ad2, We2, ae2, b2, gamma, beta, Wp, bp, Wd, bd):
    raise NotImplementedError("write your pallas kernel here")


# speedup vs baseline: 19.6992x; 19.6992x over previous
import functools
import jax
import jax.numpy as jnp
from jax import lax
from jax.experimental import pallas as pl
from jax.experimental.pallas import tpu as pltpu
from jax.experimental.pallas import tpu_sc as plsc

N = 50000
E = 800000
D = 70
ED = 4
H = 4
C = 32
NP = 50176
BN = 512
WIN = 128
NWIN = E // WIN
STRIPE = NP // 16
ZB = 64

f32 = jnp.float32
i32 = jnp.int32


def _tc_h_body(x_ref, w_ref, o_ref):
    o_ref[0] = jnp.dot(x_ref[...], w_ref[0], preferred_element_type=f32)


def _tc_h(xp, W, heads, c):
    return pl.pallas_call(
        _tc_h_body,
        grid=(heads, NP // BN),
        in_specs=[
            pl.BlockSpec((BN, xp.shape[1]), lambda h, i: (i, 0)),
            pl.BlockSpec((1, xp.shape[1], c), lambda h, i: (h, 0, 0)),
        ],
        out_specs=pl.BlockSpec((1, BN, c), lambda h, i: (h, i, 0)),
        out_shape=jax.ShapeDtypeStruct((heads, NP, c), f32),
    )(xp, W)


def _tc_sd_body(x_ref, w_ref, o_ref):
    o_ref[...] = jnp.dot(x_ref[...], w_ref[...], preferred_element_type=f32)


def _tc_sd_body2(x_ref, ws_ref, wd_ref, os_ref, od_ref):
    x = x_ref[...]
    os_ref[...] = jnp.dot(x, ws_ref[...], preferred_element_type=f32)
    od_ref[...] = jnp.dot(x, wd_ref[...], preferred_element_type=f32)


def _tc_sd(xp, Ws, Wd):
    nh = Ws.shape[1]
    return pl.pallas_call(
        _tc_sd_body2,
        grid=(NP // BN,),
        in_specs=[
            pl.BlockSpec((BN, xp.shape[1]), lambda i: (i, 0)),
            pl.BlockSpec((xp.shape[1], nh), lambda i: (0, 0)),
            pl.BlockSpec((xp.shape[1], nh), lambda i: (0, 0)),
        ],
        out_specs=[
            pl.BlockSpec((BN, nh), lambda i: (i, 0)),
            pl.BlockSpec((BN, nh), lambda i: (i, 0)),
        ],
        out_shape=[
            jax.ShapeDtypeStruct((NP, nh), f32),
            jax.ShapeDtypeStruct((NP, nh), f32),
        ],
    )(xp, Ws, Wd)


def _gelu(x):
    return 0.5 * x * (1.0 + lax.erf(x * 0.7071067811865475))


def _tc_mid_body(num_ref, den_ref, b1_ref, w2_ref, wsd_ref, h2_ref,
                 as_ref, ad_ref):
    num = num_ref[...]
    den = den_ref[...]
    x2 = _gelu(num / (den[:, :, None] + 1e-16) + b1_ref[...][:, None, :])
    acc = jnp.zeros((BN, C), f32)
    sd = jnp.zeros((BN, 2), f32)
    for hd in range(H):
        acc = acc + jnp.dot(x2[hd], w2_ref[hd], preferred_element_type=f32)
        sd = sd + jnp.dot(x2[hd], wsd_ref[hd], preferred_element_type=f32)
    h2_ref[...] = acc
    as_ref[...] = sd[:, 0:1]
    ad_ref[...] = sd[:, 1:2]


def _tc_mid(num1, den1, b1r, W2r, Wsd2r):
    return pl.pallas_call(
        _tc_mid_body,
        grid=(NP // BN,),
        in_specs=[
            pl.BlockSpec((H, BN, C), lambda i: (0, i, 0)),
            pl.BlockSpec((H, BN), lambda i: (0, i)),
            pl.BlockSpec((H, C), lambda i: (0, 0)),
            pl.BlockSpec((H, C, C), lambda i: (0, 0, 0)),
            pl.BlockSpec((H, C, 2), lambda i: (0, 0, 0)),
        ],
        out_specs=[
            pl.BlockSpec((BN, C), lambda i: (i, 0)),
            pl.BlockSpec((BN, 1), lambda i: (i, 0)),
            pl.BlockSpec((BN, 1), lambda i: (i, 0)),
        ],
        out_shape=[
            jax.ShapeDtypeStruct((NP, C), f32),
            jax.ShapeDtypeStruct((NP, 1), f32),
            jax.ShapeDtypeStruct((NP, 1), f32),
        ],
    )(num1, den1, b1r, W2r, Wsd2r)


def _tc_fin_body(num_ref, den_ref, b2_ref, g_ref, be_ref, wp_ref, bp_ref,
                 wd_ref, bd_ref, op_ref, od_ref):
    s = num_ref[0] + num_ref[1]
    d = den_ref[0] + den_ref[1]
    x3 = _gelu(s / (d[:, None] + 1e-16) + b2_ref[...])
    mu = jnp.mean(x3, axis=-1, keepdims=True)
    xc = x3 - mu
    var = jnp.mean(xc * xc, axis=-1, keepdims=True)
    xn = xc / jnp.sqrt(var + 1e-5) * g_ref[...] + be_ref[...]
    op_ref[...] = jnp.dot(xn, wp_ref[...], preferred_element_type=f32) + bp_ref[...]
    od_ref[...] = jnp.dot(xn, wd_ref[...], preferred_element_type=f32) + bd_ref[...]


def _tc_fin(num2, den2, b2r, gr, ber, Wp, bpr, Wd, bdr):
    return pl.pallas_call(
        _tc_fin_body,
        grid=(NP // BN,),
        in_specs=[
            pl.BlockSpec((2, BN, C), lambda i: (0, i, 0)),
            pl.BlockSpec((2, BN), lambda i: (0, i)),
            pl.BlockSpec((1, C), lambda i: (0, 0)),
            pl.BlockSpec((1, C), lambda i: (0, 0)),
            pl.BlockSpec((1, C), lambda i: (0, 0)),
            pl.BlockSpec((C, 8), lambda i: (0, 0)),
            pl.BlockSpec((1, 8), lambda i: (0, 0)),
            pl.BlockSpec((C, 4), lambda i: (0, 0)),
            pl.BlockSpec((1, 4), lambda i: (0, 0)),
        ],
        out_specs=[
            pl.BlockSpec((BN, 8), lambda i: (i, 0)),
            pl.BlockSpec((BN, 4), lambda i: (i, 0)),
        ],
        out_shape=[
            jax.ShapeDtypeStruct((NP, 8), f32),
            jax.ShapeDtypeStruct((NP, 4), f32),
        ],
    )(num2, den2, b2r, gr, ber, Wp, bpr, Wd, bdr)


_MESH = plsc.VectorSubcoreMesh(core_axis_name="c", subcore_axis_name="s")


def _iota16():
    return lax.iota(i32, 16)


def _full16(v):
    return jnp.full((16,), v, i32)


def _make_p1(n_heads):
    @functools.partial(
        pl.kernel,
        mesh=_MESH,
        compiler_params=pltpu.CompilerParams(needs_layout_passes=False, use_tc_tiling_on_sc=False),
        out_type=jax.ShapeDtypeStruct((n_heads * E,), f32),
        scratch_types=[
            pltpu.VMEM((WIN,), i32),
            pltpu.VMEM((WIN,), i32),
            pltpu.VMEM((WIN,), i32),
            pltpu.VMEM((WIN,), i32),
            pltpu.VMEM((WIN * ED,), f32),
            pltpu.VMEM((WIN,), f32),
            pltpu.VMEM((WIN,), f32),
            pltpu.VMEM((32,), f32),
            pltpu.VMEM((n_heads * WIN,), f32),
            pltpu.SemaphoreType.DMA,
            pltpu.SemaphoreType.DMA,
        ],
    )
    def p1(src_hbm, dst_hbm, ea_hbm, as_hbm, ad_hbm, wea_hbm, w_hbm,
           srcv, dstv, gis, gid, eav, asv, adv, weav, wbuf, sem_a, sem_b):
        cc = lax.axis_index("c")
        ss = lax.axis_index("s")
        wid = ss * 2 + cc
        nwin_my = (NWIN - wid + 31) // 32

        pltpu.sync_copy(wea_hbm, weav)
        wea_bc = [[plsc.load_gather(weav, [_full16(16 + d * 4 + hd)])
                   for hd in range(n_heads)] for d in range(ED)]

        def window(j, _):
            base = pl.multiple_of((j * 32 + wid) * WIN, WIN)
            pltpu.sync_copy(src_hbm.at[pl.ds(base, WIN)], srcv)
            pltpu.sync_copy(dst_hbm.at[pl.ds(base, WIN)], dstv)
            pltpu.sync_copy(
                ea_hbm.at[pl.ds(pl.multiple_of(base * ED, WIN), WIN * ED)], eav)
            for hd in range(n_heads):
                for g in range(WIN // 16):
                    sl = pl.ds(g * 16, 16)
                    gis[sl] = srcv[sl] * n_heads + hd
                    gid[sl] = dstv[sl] * n_heads + hd
                ga = pltpu.async_copy(as_hbm.at[gis], asv, sem_a)
                gb = pltpu.async_copy(ad_hbm.at[gid], adv, sem_b)
                ga.wait()
                gb.wait()
                for g in range(WIN // 16):
                    sl = pl.ds(g * 16, 16)
                    eids = _iota16() + g * 16
                    al = asv[sl] + adv[sl]
                    for d in range(ED):
                        attr = plsc.load_gather(eav, [eids * ED + d])
                        al = al + attr * wea_bc[d][hd]
                    al = jnp.where(al >= 0.0, al, al * 0.2)
                    wbuf[pl.ds(hd * WIN + g * 16, 16)] = jnp.exp(al)
            for hd in range(n_heads):
                pltpu.sync_copy(
                    wbuf.at[pl.ds(hd * WIN, WIN)],
                    w_hbm.at[pl.ds(pl.multiple_of(hd * E + base, WIN), WIN)])
            return 0

        lax.fori_loop(0, nwin_my, window, 0)

    return p1


_p1_l1 = _make_p1(H)
_p1_l2 = _make_p1(1)


def _make_p2(layer):
    if layer == 1:
        n_pass, tab_rows, out_rows = 2, H * NP, H
    else:
        n_pass, tab_rows, out_rows = 1, NP, 2

    @functools.partial(
        pl.kernel,
        mesh=_MESH,
        compiler_params=pltpu.CompilerParams(needs_layout_passes=False, use_tc_tiling_on_sc=False),
        out_type=[
            jax.ShapeDtypeStruct((out_rows * NP, C), f32),
            jax.ShapeDtypeStruct((out_rows * NP,), f32),
        ],
        scratch_types=[
            pltpu.VMEM((WIN,), i32),
            pltpu.VMEM((WIN,), i32),
            pltpu.VMEM((WIN,), i32),
            pltpu.VMEM((16 + WIN,), f32),
            pltpu.VMEM((WIN, C), f32),
            pltpu.VMEM((ZB, C), f32),
            pltpu.VMEM((ZB,), f32),
            pltpu.SemaphoreType.DMA,
            pltpu.VMEM_SHARED((NP, C), f32),
            pltpu.VMEM_SHARED((NP,), f32),
        ],
    )
    def p2(src_hbm, dst_hbm, w_hbm, h_hbm, num_hbm, den_hbm,
           srcv, dstv, gidx, wv, rows, zb, zv, sem, nums, dens):
        cc = lax.axis_index("c")
        ss = lax.axis_index("s")

        for i in range(ZB):
            zb[i, pl.ds(0, 16)] = jnp.zeros((16,), f32)
            zb[i, pl.ds(16, 16)] = jnp.zeros((16,), f32)
        for q in range(ZB // 16):
            zv[pl.ds(q * 16, 16)] = jnp.zeros((16,), f32)

        sbase = ss * STRIPE

        for k in range(n_pass):
            if layer == 1:
                hd = cc * 2 + k
                win0 = 0
                nwin_sc = NWIN
                orow = hd
            else:
                hd = jnp.zeros((), i32)
                win0 = cc * (NWIN // 2)
                nwin_sc = NWIN // 2
                orow = cc

            def zcp(b, _):
                pltpu.sync_copy(zb, nums.at[pl.ds(sbase + b * ZB, ZB)])
                pltpu.sync_copy(zv, dens.at[pl.ds(sbase + b * ZB, ZB)])
                return 0

            lax.fori_loop(0, STRIPE // ZB, zcp, 0)
            plsc.subcore_barrier()

            nwin_my = (nwin_sc - ss + 15) // 16

            def window(j, _):
                base = pl.multiple_of((win0 + j * 16 + ss) * WIN, WIN)
                pltpu.sync_copy(src_hbm.at[pl.ds(base, WIN)], srcv)
                pltpu.sync_copy(dst_hbm.at[pl.ds(base, WIN)], dstv)
                pltpu.sync_copy(
                    w_hbm.at[pl.ds(pl.multiple_of(hd * E + base, WIN), WIN)],
                    wv.at[pl.ds(16, WIN)])
                off = hd * NP
                for g in range(WIN // 16):
                    sl = pl.ds(g * 16, 16)
                    gidx[sl] = srcv[sl] + off
                pltpu.async_copy(h_hbm.at[gidx], rows, sem).wait()

                for i in range(WIN):
                    wb = plsc.load_gather(wv, [_full16(16 + i)])
                    rows[i, pl.ds(0, 16)] = rows[i, pl.ds(0, 16)] * wb
                    rows[i, pl.ds(16, 16)] = rows[i, pl.ds(16, 16)] * wb
                pltpu.sync_copy(rows, nums.at[dstv], add=True)
                pltpu.sync_copy(wv.at[pl.ds(16, WIN)], dens.at[dstv], add=True)
                return 0

            lax.fori_loop(0, nwin_my, window, 0)
            plsc.subcore_barrier()

            pltpu.sync_copy(
                nums.at[pl.ds(sbase, STRIPE)],
                num_hbm.at[pl.ds(pl.multiple_of(orow * NP + sbase, ZB), STRIPE)])
            pltpu.sync_copy(
                dens.at[pl.ds(sbase, STRIPE)],
                den_hbm.at[pl.ds(pl.multiple_of(orow * NP + sbase, ZB), STRIPE)])
            if k + 1 < n_pass:
                plsc.subcore_barrier()

    return p2


_p2_l1 = _make_p2(1)
_p2_l2 = _make_p2(2)


def _fold(W, av, heads, c):
    return (W.reshape(-1, heads, c) * av[0][None]).sum(-1)


def kernel(node_features, edge_index, edge_attr, W1, as1, ad1, We1, ae1, b1,
           W2, as2, ad2, We2, ae2, b2, gamma, beta, Wp, bp, Wd, bd):
    src = edge_index[0].astype(i32)
    dst = edge_index[1].astype(i32)
    xp = jnp.pad(node_features, ((0, NP - N), (0, 0)))

    Was1 = _fold(W1, as1, H, C)
    Wad1 = _fold(W1, ad1, H, C)
    z16 = jnp.zeros((16,), f32)
    wea1 = jnp.concatenate(
        [z16, jnp.pad(_fold(We1, ae1, H, C), ((0, 0), (0, 4 - H))).reshape(16)])
    Wsd2 = jnp.concatenate(
        [_fold(W2, as2, 1, C), _fold(W2, ad2, 1, C)], axis=1)
    wea2 = jnp.concatenate(
        [z16, jnp.pad(_fold(We2, ae2, 1, C), ((0, 0), (0, 3))).reshape(16)])
    ea_flat = edge_attr.reshape(E * ED)

    W1h = W1.reshape(D, H, C).transpose(1, 0, 2)
    h1 = _tc_h(xp, W1h, H, C).reshape(H * NP, C)
    as1t, ad1t = _tc_sd(xp, Was1, Wad1)
    w1 = _p1_l1(src, dst, ea_flat, as1t.reshape(-1), ad1t.reshape(-1), wea1)
    num1, den1 = _p2_l1(src, dst, w1, h1)

    b1r = b1.reshape(H, C)
    W2r = W2.reshape(H, C, C)
    Wsd2r = Wsd2.reshape(H, C, 2)
    h2, as2t, ad2t = _tc_mid(num1.reshape(H, NP, C), den1.reshape(H, NP),
                             b1r, W2r, Wsd2r)

    w2 = _p1_l2(src, dst, ea_flat, as2t.reshape(-1), ad2t.reshape(-1), wea2)
    num2, den2 = _p2_l2(src, dst, w2, h2)

    op, od = _tc_fin(num2.reshape(2, NP, C), den2.reshape(2, NP),
                     b2.reshape(1, C), gamma.reshape(1, C), beta.reshape(1, C),
                     Wp, bp.reshape(1, 8), Wd, bd.reshape(1, 4))
    return (op[:N], od[:N])


# speedup vs baseline: 26.2524x; 1.3327x over previous
import functools
import jax
import jax.numpy as jnp
from jax import lax
from jax.experimental import pallas as pl
from jax.experimental.pallas import tpu as pltpu
from jax.experimental.pallas import tpu_sc as plsc

N = 50000
E = 800000
D = 70
ED = 4
H = 4
C = 32
NP = 50176
BN = 512
WIN = 128
NWIN = E // WIN
STRIPE = NP // 16
ZB = 64

f32 = jnp.float32
i32 = jnp.int32


def _tc_h_body(x_ref, w_ref, o_ref):
    o_ref[0] = jnp.dot(x_ref[...], w_ref[0], preferred_element_type=f32)


def _tc_h(xp, W, heads, c):
    return pl.pallas_call(
        _tc_h_body,
        grid=(heads, NP // BN),
        in_specs=[
            pl.BlockSpec((BN, xp.shape[1]), lambda h, i: (i, 0)),
            pl.BlockSpec((1, xp.shape[1], c), lambda h, i: (h, 0, 0)),
        ],
        out_specs=pl.BlockSpec((1, BN, c), lambda h, i: (h, i, 0)),
        out_shape=jax.ShapeDtypeStruct((heads, NP, c), f32),
    )(xp, W)


def _tc_sd_body(x_ref, w_ref, o_ref):
    o_ref[...] = jnp.dot(x_ref[...], w_ref[...], preferred_element_type=f32)


def _tc_sd_body2(x_ref, ws_ref, wd_ref, os_ref, od_ref):
    x = x_ref[...]
    os_ref[...] = jnp.dot(x, ws_ref[...], preferred_element_type=f32)
    od_ref[...] = jnp.dot(x, wd_ref[...], preferred_element_type=f32)


def _tc_sd(xp, Ws, Wd):
    nh = Ws.shape[1]
    return pl.pallas_call(
        _tc_sd_body2,
        grid=(NP // BN,),
        in_specs=[
            pl.BlockSpec((BN, xp.shape[1]), lambda i: (i, 0)),
            pl.BlockSpec((xp.shape[1], nh), lambda i: (0, 0)),
            pl.BlockSpec((xp.shape[1], nh), lambda i: (0, 0)),
        ],
        out_specs=[
            pl.BlockSpec((BN, nh), lambda i: (i, 0)),
            pl.BlockSpec((BN, nh), lambda i: (i, 0)),
        ],
        out_shape=[
            jax.ShapeDtypeStruct((NP, nh), f32),
            jax.ShapeDtypeStruct((NP, nh), f32),
        ],
    )(xp, Ws, Wd)


def _gelu(x):
    return 0.5 * x * (1.0 + lax.erf(x * 0.7071067811865475))


def _tc_mid_body(num_ref, den_ref, b1_ref, w2_ref, wsd_ref, h2_ref,
                 as_ref, ad_ref):
    num = num_ref[...]
    den = den_ref[...]
    x2 = _gelu(num / (den[:, :, None] + 1e-16) + b1_ref[...][:, None, :])
    acc = jnp.zeros((BN, C), f32)
    sd = jnp.zeros((BN, 2), f32)
    for hd in range(H):
        acc = acc + jnp.dot(x2[hd], w2_ref[hd], preferred_element_type=f32)
        sd = sd + jnp.dot(x2[hd], wsd_ref[hd], preferred_element_type=f32)
    h2_ref[...] = acc
    as_ref[...] = sd[:, 0:1]
    ad_ref[...] = sd[:, 1:2]


def _tc_mid(num1, den1, b1r, W2r, Wsd2r):
    return pl.pallas_call(
        _tc_mid_body,
        grid=(NP // BN,),
        in_specs=[
            pl.BlockSpec((H, BN, C), lambda i: (0, i, 0)),
            pl.BlockSpec((H, BN), lambda i: (0, i)),
            pl.BlockSpec((H, C), lambda i: (0, 0)),
            pl.BlockSpec((H, C, C), lambda i: (0, 0, 0)),
            pl.BlockSpec((H, C, 2), lambda i: (0, 0, 0)),
        ],
        out_specs=[
            pl.BlockSpec((BN, C), lambda i: (i, 0)),
            pl.BlockSpec((BN, 1), lambda i: (i, 0)),
            pl.BlockSpec((BN, 1), lambda i: (i, 0)),
        ],
        out_shape=[
            jax.ShapeDtypeStruct((NP, C), f32),
            jax.ShapeDtypeStruct((NP, 1), f32),
            jax.ShapeDtypeStruct((NP, 1), f32),
        ],
    )(num1, den1, b1r, W2r, Wsd2r)


def _tc_fin_body(num_ref, den_ref, b2_ref, g_ref, be_ref, wp_ref, bp_ref,
                 wd_ref, bd_ref, op_ref, od_ref):
    s = num_ref[0] + num_ref[1]
    d = den_ref[0] + den_ref[1]
    x3 = _gelu(s / (d[:, None] + 1e-16) + b2_ref[...])
    mu = jnp.mean(x3, axis=-1, keepdims=True)
    xc = x3 - mu
    var = jnp.mean(xc * xc, axis=-1, keepdims=True)
    xn = xc / jnp.sqrt(var + 1e-5) * g_ref[...] + be_ref[...]
    op_ref[...] = jnp.dot(xn, wp_ref[...], preferred_element_type=f32) + bp_ref[...]
    od_ref[...] = jnp.dot(xn, wd_ref[...], preferred_element_type=f32) + bd_ref[...]


def _tc_fin(num2, den2, b2r, gr, ber, Wp, bpr, Wd, bdr):
    return pl.pallas_call(
        _tc_fin_body,
        grid=(NP // BN,),
        in_specs=[
            pl.BlockSpec((2, BN, C), lambda i: (0, i, 0)),
            pl.BlockSpec((2, BN), lambda i: (0, i)),
            pl.BlockSpec((1, C), lambda i: (0, 0)),
            pl.BlockSpec((1, C), lambda i: (0, 0)),
            pl.BlockSpec((1, C), lambda i: (0, 0)),
            pl.BlockSpec((C, 8), lambda i: (0, 0)),
            pl.BlockSpec((1, 8), lambda i: (0, 0)),
            pl.BlockSpec((C, 4), lambda i: (0, 0)),
            pl.BlockSpec((1, 4), lambda i: (0, 0)),
        ],
        out_specs=[
            pl.BlockSpec((BN, 8), lambda i: (i, 0)),
            pl.BlockSpec((BN, 4), lambda i: (i, 0)),
        ],
        out_shape=[
            jax.ShapeDtypeStruct((NP, 8), f32),
            jax.ShapeDtypeStruct((NP, 4), f32),
        ],
    )(num2, den2, b2r, gr, ber, Wp, bpr, Wd, bdr)


_MESH = plsc.VectorSubcoreMesh(core_axis_name="c", subcore_axis_name="s")


def _iota16():
    return lax.iota(i32, 16)


def _full16(v):
    return jnp.full((16,), v, i32)


def _make_p1(n_heads):
    @functools.partial(
        pl.kernel,
        mesh=_MESH,
        compiler_params=pltpu.CompilerParams(needs_layout_passes=False, use_tc_tiling_on_sc=False),
        out_type=jax.ShapeDtypeStruct((n_heads * E,), f32),
        scratch_types=[
            pltpu.VMEM((WIN,), i32),
            pltpu.VMEM((WIN,), i32),
            pltpu.VMEM((WIN * n_heads,), i32),
            pltpu.VMEM((WIN * n_heads,), i32),
            pltpu.VMEM((WIN * ED,), f32),
            pltpu.VMEM((WIN * n_heads,), f32),
            pltpu.VMEM((WIN * n_heads,), f32),
            pltpu.VMEM((32,), f32),
            pltpu.VMEM((n_heads * WIN,), f32),
            pltpu.SemaphoreType.DMA,
            pltpu.SemaphoreType.DMA,
            pltpu.SemaphoreType.DMA,
            pltpu.SemaphoreType.DMA,
            pltpu.SemaphoreType.DMA,
        ],
    )
    def p1(src_hbm, dst_hbm, ea_hbm, as_hbm, ad_hbm, wea_hbm, w_hbm,
           srcv, dstv, gis, gid, eav, asv, adv, weav, wbuf,
           sem_a, sem_b, sem_c, sem_d, sem_e):
        cc = lax.axis_index("c")
        ss = lax.axis_index("s")
        wid = ss * 2 + cc
        nwin_my = (NWIN - wid + 31) // 32

        pltpu.sync_copy(wea_hbm, weav)
        wea_bc = [[plsc.load_gather(weav, [_full16(16 + d * 4 + hd)])
                   for hd in range(n_heads)] for d in range(ED)]

        def window(j, _):
            base = pl.multiple_of((j * 32 + wid) * WIN, WIN)
            d1 = pltpu.async_copy(src_hbm.at[pl.ds(base, WIN)], srcv, sem_a)
            d2 = pltpu.async_copy(dst_hbm.at[pl.ds(base, WIN)], dstv, sem_b)
            d3 = pltpu.async_copy(
                ea_hbm.at[pl.ds(pl.multiple_of(base * ED, WIN), WIN * ED)],
                eav, sem_c)
            d1.wait()
            d2.wait()
            if n_heads > 1:
                for g in range(WIN * n_heads // 16):
                    sl = pl.ds(g * 16, 16)
                    q = lax.shift_right_logical(_iota16(), 2) + g * 4
                    rem = jnp.bitwise_and(_iota16(), 3)
                    sv = plsc.load_gather(srcv, [q]) * n_heads + rem
                    dv = plsc.load_gather(dstv, [q]) * n_heads + rem
                    gis[sl] = sv
                    gid[sl] = dv
            else:
                for g in range(WIN // 16):
                    sl = pl.ds(g * 16, 16)
                    gis[sl] = srcv[sl]
                    gid[sl] = dstv[sl]
            ga = pltpu.async_copy(as_hbm.at[gis], asv, sem_d)
            gb = pltpu.async_copy(ad_hbm.at[gid], adv, sem_e)
            d3.wait()
            ga.wait()
            gb.wait()
            for hd in range(n_heads):
                for g in range(WIN // 16):
                    eids = _iota16() + g * 16
                    if n_heads > 1:
                        al = (plsc.load_gather(asv, [eids * n_heads + hd])
                              + plsc.load_gather(adv, [eids * n_heads + hd]))
                    else:
                        sl = pl.ds(g * 16, 16)
                        al = asv[sl] + adv[sl]
                    for d in range(ED):
                        attr = plsc.load_gather(eav, [eids * ED + d])
                        al = al + attr * wea_bc[d][hd]
                    al = jnp.where(al >= 0.0, al, al * 0.2)
                    wbuf[pl.ds(hd * WIN + g * 16, 16)] = jnp.exp(al)
            for hd in range(n_heads):
                pltpu.sync_copy(
                    wbuf.at[pl.ds(hd * WIN, WIN)],
                    w_hbm.at[pl.ds(pl.multiple_of(hd * E + base, WIN), WIN)])
            return 0

        lax.fori_loop(0, nwin_my, window, 0)

    return p1


_p1_l1 = _make_p1(H)
_p1_l2 = _make_p1(1)


def _make_p2(layer):
    if layer == 1:
        n_pass, tab_rows, out_rows = 2, H * NP, H
    else:
        n_pass, tab_rows, out_rows = 1, NP, 2

    @functools.partial(
        pl.kernel,
        mesh=_MESH,
        compiler_params=pltpu.CompilerParams(needs_layout_passes=False, use_tc_tiling_on_sc=False),
        out_type=[
            jax.ShapeDtypeStruct((out_rows * NP, C), f32),
            jax.ShapeDtypeStruct((out_rows * NP,), f32),
        ],
        scratch_types=[
            pltpu.VMEM((WIN,), i32),
            pltpu.VMEM((WIN,), i32),
            pltpu.VMEM((WIN,), i32),
            pltpu.VMEM((16 + WIN,), f32),
            pltpu.VMEM((WIN, C), f32),
            pltpu.VMEM((ZB, C), f32),
            pltpu.VMEM((ZB,), f32),
            pltpu.SemaphoreType.DMA,
            pltpu.SemaphoreType.DMA,
            pltpu.SemaphoreType.DMA,
            pltpu.SemaphoreType.DMA,
            pltpu.VMEM_SHARED((NP, C), f32),
            pltpu.VMEM_SHARED((NP,), f32),
        ],
    )
    def p2(src_hbm, dst_hbm, w_hbm, h_hbm, num_hbm, den_hbm,
           srcv, dstv, gidx, wv, rows, zb, zv, sem, sem_b, sem_c, sem_d,
           nums, dens):
        cc = lax.axis_index("c")
        ss = lax.axis_index("s")

        for i in range(ZB):
            zb[i, pl.ds(0, 16)] = jnp.zeros((16,), f32)
            zb[i, pl.ds(16, 16)] = jnp.zeros((16,), f32)
        for q in range(ZB // 16):
            zv[pl.ds(q * 16, 16)] = jnp.zeros((16,), f32)

        sbase = ss * STRIPE

        for k in range(n_pass):
            if layer == 1:
                hd = cc * 2 + k
                win0 = 0
                nwin_sc = NWIN
                orow = hd
            else:
                hd = jnp.zeros((), i32)
                win0 = cc * (NWIN // 2)
                nwin_sc = NWIN // 2
                orow = cc

            def zcp(b, _):
                pltpu.sync_copy(zb, nums.at[pl.ds(sbase + b * ZB, ZB)])
                pltpu.sync_copy(zv, dens.at[pl.ds(sbase + b * ZB, ZB)])
                return 0

            lax.fori_loop(0, STRIPE // ZB, zcp, 0)
            plsc.subcore_barrier()

            nwin_my = (nwin_sc - ss + 15) // 16

            def window(j, _):
                base = pl.multiple_of((win0 + j * 16 + ss) * WIN, WIN)
                d1 = pltpu.async_copy(src_hbm.at[pl.ds(base, WIN)], srcv, sem)
                d2 = pltpu.async_copy(dst_hbm.at[pl.ds(base, WIN)], dstv, sem_b)
                d3 = pltpu.async_copy(
                    w_hbm.at[pl.ds(pl.multiple_of(hd * E + base, WIN), WIN)],
                    wv.at[pl.ds(16, WIN)], sem_c)
                d1.wait()
                off = hd * NP
                for g in range(WIN // 16):
                    sl = pl.ds(g * 16, 16)
                    gidx[sl] = srcv[sl] + off
                d4 = pltpu.async_copy(h_hbm.at[gidx], rows, sem_d)
                d3.wait()
                d4.wait()

                for i in range(WIN):
                    wb = plsc.load_gather(wv, [_full16(16 + i)])
                    rows[i, pl.ds(0, 16)] = rows[i, pl.ds(0, 16)] * wb
                    rows[i, pl.ds(16, 16)] = rows[i, pl.ds(16, 16)] * wb
                d2.wait()
                pltpu.sync_copy(rows, nums.at[dstv], add=True)
                pltpu.sync_copy(wv.at[pl.ds(16, WIN)], dens.at[dstv], add=True)
                return 0

            lax.fori_loop(0, nwin_my, window, 0)
            plsc.subcore_barrier()

            pltpu.sync_copy(
                nums.at[pl.ds(sbase, STRIPE)],
                num_hbm.at[pl.ds(pl.multiple_of(orow * NP + sbase, ZB), STRIPE)])
            pltpu.sync_copy(
                dens.at[pl.ds(sbase, STRIPE)],
                den_hbm.at[pl.ds(pl.multiple_of(orow * NP + sbase, ZB), STRIPE)])
            if k + 1 < n_pass:
                plsc.subcore_barrier()

    return p2


_p2_l1 = _make_p2(1)
_p2_l2 = _make_p2(2)


def _fold(W, av, heads, c):
    return (W.reshape(-1, heads, c) * av[0][None]).sum(-1)


def kernel(node_features, edge_index, edge_attr, W1, as1, ad1, We1, ae1, b1,
           W2, as2, ad2, We2, ae2, b2, gamma, beta, Wp, bp, Wd, bd):
    src = edge_index[0].astype(i32)
    dst = edge_index[1].astype(i32)
    xp = jnp.pad(node_features, ((0, NP - N), (0, 0)))

    Was1 = _fold(W1, as1, H, C)
    Wad1 = _fold(W1, ad1, H, C)
    z16 = jnp.zeros((16,), f32)
    wea1 = jnp.concatenate(
        [z16, jnp.pad(_fold(We1, ae1, H, C), ((0, 0), (0, 4 - H))).reshape(16)])
    Wsd2 = jnp.concatenate(
        [_fold(W2, as2, 1, C), _fold(W2, ad2, 1, C)], axis=1)
    wea2 = jnp.concatenate(
        [z16, jnp.pad(_fold(We2, ae2, 1, C), ((0, 0), (0, 3))).reshape(16)])
    ea_flat = edge_attr.reshape(E * ED)

    W1h = W1.reshape(D, H, C).transpose(1, 0, 2)
    h1 = _tc_h(xp, W1h, H, C).reshape(H * NP, C)
    as1t, ad1t = _tc_sd(xp, Was1, Wad1)
    w1 = _p1_l1(src, dst, ea_flat, as1t.reshape(-1), ad1t.reshape(-1), wea1)
    num1, den1 = _p2_l1(src, dst, w1, h1)

    b1r = b1.reshape(H, C)
    W2r = W2.reshape(H, C, C)
    Wsd2r = Wsd2.reshape(H, C, 2)
    h2, as2t, ad2t = _tc_mid(num1.reshape(H, NP, C), den1.reshape(H, NP),
                             b1r, W2r, Wsd2r)

    w2 = _p1_l2(src, dst, ea_flat, as2t.reshape(-1), ad2t.reshape(-1), wea2)
    num2, den2 = _p2_l2(src, dst, w2, h2)

    op, od = _tc_fin(num2.reshape(2, NP, C), den2.reshape(2, NP),
                     b2.reshape(1, C), gamma.reshape(1, C), beta.reshape(1, C),
                     Wp, bp.reshape(1, 8), Wd, bd.reshape(1, 4))
    return (op[:N], od[:N])


# speedup vs baseline: 26.8297x; 1.0220x over previous
import functools
import jax
import jax.numpy as jnp
from jax import lax
from jax.experimental import pallas as pl
from jax.experimental.pallas import tpu as pltpu
from jax.experimental.pallas import tpu_sc as plsc

N = 50000
E = 800000
D = 70
ED = 4
H = 4
C = 32
NP = 50176
BN = 512
WIN = 128
NWIN = E // WIN
STRIPE = NP // 16
ZB = 64

f32 = jnp.float32
i32 = jnp.int32


def _tc_h_body(x_ref, w_ref, o_ref):
    o_ref[0] = jnp.dot(x_ref[...], w_ref[0], preferred_element_type=f32)


def _tc_h(xp, W, heads, c):
    return pl.pallas_call(
        _tc_h_body,
        grid=(heads, NP // BN),
        in_specs=[
            pl.BlockSpec((BN, xp.shape[1]), lambda h, i: (i, 0)),
            pl.BlockSpec((1, xp.shape[1], c), lambda h, i: (h, 0, 0)),
        ],
        out_specs=pl.BlockSpec((1, BN, c), lambda h, i: (h, i, 0)),
        out_shape=jax.ShapeDtypeStruct((heads, NP, c), f32),
    )(xp, W)


def _tc_sd_body(x_ref, w_ref, o_ref):
    o_ref[...] = jnp.dot(x_ref[...], w_ref[...], preferred_element_type=f32)


def _tc_sd_body2(x_ref, ws_ref, wd_ref, os_ref, od_ref):
    x = x_ref[...]
    os_ref[...] = jnp.dot(x, ws_ref[...], preferred_element_type=f32)
    od_ref[...] = jnp.dot(x, wd_ref[...], preferred_element_type=f32)


def _tc_sd(xp, Ws, Wd):
    nh = Ws.shape[1]
    return pl.pallas_call(
        _tc_sd_body2,
        grid=(NP // BN,),
        in_specs=[
            pl.BlockSpec((BN, xp.shape[1]), lambda i: (i, 0)),
            pl.BlockSpec((xp.shape[1], nh), lambda i: (0, 0)),
            pl.BlockSpec((xp.shape[1], nh), lambda i: (0, 0)),
        ],
        out_specs=[
            pl.BlockSpec((BN, nh), lambda i: (i, 0)),
            pl.BlockSpec((BN, nh), lambda i: (i, 0)),
        ],
        out_shape=[
            jax.ShapeDtypeStruct((NP, nh), f32),
            jax.ShapeDtypeStruct((NP, nh), f32),
        ],
    )(xp, Ws, Wd)


def _gelu(x):
    return 0.5 * x * (1.0 + lax.erf(x * 0.7071067811865475))


def _tc_mid_body(num_ref, den_ref, b1_ref, w2_ref, wsd_ref, h2_ref,
                 as_ref, ad_ref):
    num = num_ref[...]
    den = den_ref[...]
    x2 = _gelu(num / (den[:, :, None] + 1e-16) + b1_ref[...][:, None, :])
    acc = jnp.zeros((BN, C), f32)
    sd = jnp.zeros((BN, 2), f32)
    for hd in range(H):
        acc = acc + jnp.dot(x2[hd], w2_ref[hd], preferred_element_type=f32)
        sd = sd + jnp.dot(x2[hd], wsd_ref[hd], preferred_element_type=f32)
    h2_ref[...] = acc
    as_ref[...] = sd[:, 0:1]
    ad_ref[...] = sd[:, 1:2]


def _tc_mid(num1, den1, b1r, W2r, Wsd2r):
    return pl.pallas_call(
        _tc_mid_body,
        grid=(NP // BN,),
        in_specs=[
            pl.BlockSpec((H, BN, C), lambda i: (0, i, 0)),
            pl.BlockSpec((H, BN), lambda i: (0, i)),
            pl.BlockSpec((H, C), lambda i: (0, 0)),
            pl.BlockSpec((H, C, C), lambda i: (0, 0, 0)),
            pl.BlockSpec((H, C, 2), lambda i: (0, 0, 0)),
        ],
        out_specs=[
            pl.BlockSpec((BN, C), lambda i: (i, 0)),
            pl.BlockSpec((BN, 1), lambda i: (i, 0)),
            pl.BlockSpec((BN, 1), lambda i: (i, 0)),
        ],
        out_shape=[
            jax.ShapeDtypeStruct((NP, C), f32),
            jax.ShapeDtypeStruct((NP, 1), f32),
            jax.ShapeDtypeStruct((NP, 1), f32),
        ],
    )(num1, den1, b1r, W2r, Wsd2r)


def _tc_fin_body(num_ref, den_ref, b2_ref, g_ref, be_ref, wp_ref, bp_ref,
                 wd_ref, bd_ref, op_ref, od_ref):
    s = num_ref[0] + num_ref[1]
    d = den_ref[0] + den_ref[1]
    x3 = _gelu(s / (d[:, None] + 1e-16) + b2_ref[...])
    mu = jnp.mean(x3, axis=-1, keepdims=True)
    xc = x3 - mu
    var = jnp.mean(xc * xc, axis=-1, keepdims=True)
    xn = xc / jnp.sqrt(var + 1e-5) * g_ref[...] + be_ref[...]
    op_ref[...] = jnp.dot(xn, wp_ref[...], preferred_element_type=f32) + bp_ref[...]
    od_ref[...] = jnp.dot(xn, wd_ref[...], preferred_element_type=f32) + bd_ref[...]


def _tc_fin(num2, den2, b2r, gr, ber, Wp, bpr, Wd, bdr):
    return pl.pallas_call(
        _tc_fin_body,
        grid=(NP // BN,),
        in_specs=[
            pl.BlockSpec((2, BN, C), lambda i: (0, i, 0)),
            pl.BlockSpec((2, BN), lambda i: (0, i)),
            pl.BlockSpec((1, C), lambda i: (0, 0)),
            pl.BlockSpec((1, C), lambda i: (0, 0)),
            pl.BlockSpec((1, C), lambda i: (0, 0)),
            pl.BlockSpec((C, 8), lambda i: (0, 0)),
            pl.BlockSpec((1, 8), lambda i: (0, 0)),
            pl.BlockSpec((C, 4), lambda i: (0, 0)),
            pl.BlockSpec((1, 4), lambda i: (0, 0)),
        ],
        out_specs=[
            pl.BlockSpec((BN, 8), lambda i: (i, 0)),
            pl.BlockSpec((BN, 4), lambda i: (i, 0)),
        ],
        out_shape=[
            jax.ShapeDtypeStruct((NP, 8), f32),
            jax.ShapeDtypeStruct((NP, 4), f32),
        ],
    )(num2, den2, b2r, gr, ber, Wp, bpr, Wd, bdr)


_MESH = plsc.VectorSubcoreMesh(core_axis_name="c", subcore_axis_name="s")


def _iota16():
    return lax.iota(i32, 16)


def _full16(v):
    return jnp.full((16,), v, i32)


def _make_p1(n_heads):
    @functools.partial(
        pl.kernel,
        mesh=_MESH,
        compiler_params=pltpu.CompilerParams(needs_layout_passes=False, use_tc_tiling_on_sc=False),
        out_type=jax.ShapeDtypeStruct((n_heads * E,), f32),
        scratch_types=[
            pltpu.VMEM((WIN,), i32),
            pltpu.VMEM((WIN,), i32),
            pltpu.VMEM((WIN * n_heads,), i32),
            pltpu.VMEM((WIN * n_heads,), i32),
            pltpu.VMEM((WIN * ED,), f32),
            pltpu.VMEM((WIN * n_heads,), f32),
            pltpu.VMEM((WIN * n_heads,), f32),
            pltpu.VMEM((32,), f32),
            pltpu.VMEM((n_heads * WIN,), f32),
            pltpu.SemaphoreType.DMA,
            pltpu.SemaphoreType.DMA,
            pltpu.SemaphoreType.DMA,
            pltpu.SemaphoreType.DMA,
            pltpu.SemaphoreType.DMA,
        ],
    )
    def p1(src_hbm, dst_hbm, ea_hbm, as_hbm, ad_hbm, wea_hbm, w_hbm,
           srcv, dstv, gis, gid, eav, asv, adv, weav, wbuf,
           sem_a, sem_b, sem_c, sem_d, sem_e):
        cc = lax.axis_index("c")
        ss = lax.axis_index("s")
        wid = ss * 2 + cc
        nwin_my = (NWIN - wid + 31) // 32

        pltpu.sync_copy(wea_hbm, weav)
        wea_bc = [[plsc.load_gather(weav, [_full16(16 + d * 4 + hd)])
                   for hd in range(n_heads)] for d in range(ED)]

        def window(j, _):
            base = pl.multiple_of((j * 32 + wid) * WIN, WIN)
            d1 = pltpu.async_copy(src_hbm.at[pl.ds(base, WIN)], srcv, sem_a)
            d2 = pltpu.async_copy(dst_hbm.at[pl.ds(base, WIN)], dstv, sem_b)
            d3 = pltpu.async_copy(
                ea_hbm.at[pl.ds(pl.multiple_of(base * ED, WIN), WIN * ED)],
                eav, sem_c)
            d1.wait()
            d2.wait()
            if n_heads > 1:
                for g in range(WIN * n_heads // 16):
                    sl = pl.ds(g * 16, 16)
                    q = lax.shift_right_logical(_iota16(), 2) + g * 4
                    rem = jnp.bitwise_and(_iota16(), 3)
                    sv = plsc.load_gather(srcv, [q]) * n_heads + rem
                    dv = plsc.load_gather(dstv, [q]) * n_heads + rem
                    gis[sl] = sv
                    gid[sl] = dv
            else:
                for g in range(WIN // 16):
                    sl = pl.ds(g * 16, 16)
                    gis[sl] = srcv[sl]
                    gid[sl] = dstv[sl]
            ga = pltpu.async_copy(as_hbm.at[gis], asv, sem_d)
            gb = pltpu.async_copy(ad_hbm.at[gid], adv, sem_e)
            d3.wait()
            ga.wait()
            gb.wait()
            for hd in range(n_heads):
                for g in range(WIN // 16):
                    eids = _iota16() + g * 16
                    if n_heads > 1:
                        al = (plsc.load_gather(asv, [eids * n_heads + hd])
                              + plsc.load_gather(adv, [eids * n_heads + hd]))
                    else:
                        sl = pl.ds(g * 16, 16)
                        al = asv[sl] + adv[sl]
                    for d in range(ED):
                        attr = plsc.load_gather(eav, [eids * ED + d])
                        al = al + attr * wea_bc[d][hd]
                    al = jnp.where(al >= 0.0, al, al * 0.2)
                    wbuf[pl.ds(hd * WIN + g * 16, 16)] = jnp.exp(al)
            for hd in range(n_heads):
                pltpu.sync_copy(
                    wbuf.at[pl.ds(hd * WIN, WIN)],
                    w_hbm.at[pl.ds(pl.multiple_of(hd * E + base, WIN), WIN)])
            return 0

        lax.fori_loop(0, nwin_my, window, 0)

    return p1


_p1_l1 = _make_p1(H)
_p1_l2 = _make_p1(1)


def _make_p2(layer):
    if layer == 1:
        n_pass, out_rows = 2, H
    else:
        n_pass, out_rows = 1, 2
    WB = 5
    W2 = WB * WIN
    NW2 = E // W2

    @functools.partial(
        pl.kernel,
        mesh=_MESH,
        compiler_params=pltpu.CompilerParams(needs_layout_passes=False, use_tc_tiling_on_sc=False),
        out_type=[
            jax.ShapeDtypeStruct((out_rows * NP, C), f32),
            jax.ShapeDtypeStruct((out_rows * NP,), f32),
        ],
        scratch_types=[
            pltpu.VMEM((W2,), i32),
            [pltpu.VMEM((WIN,), i32) for _ in range(WB)],
            [pltpu.VMEM((WIN,), i32) for _ in range(WB)],
            pltpu.VMEM((16 + W2,), f32),
            pltpu.VMEM((W2, C), f32),
            pltpu.VMEM((ZB, C), f32),
            pltpu.VMEM((ZB,), f32),
            pltpu.SemaphoreType.DMA,
            pltpu.SemaphoreType.DMA,
            [pltpu.SemaphoreType.DMA for _ in range(WB)],
            [pltpu.SemaphoreType.DMA for _ in range(WB)],
            pltpu.VMEM_SHARED((NP, C), f32),
            pltpu.VMEM_SHARED((NP,), f32),
        ],
    )
    def p2(src_hbm, dst_hbm, w_hbm, h_hbm, num_hbm, den_hbm,
           srcv, dstvs, gidxs, wv, rows, zb, zv, sem, sem_c, sem_ds, sem_gs,
           nums, dens):
        cc = lax.axis_index("c")
        ss = lax.axis_index("s")

        for i in range(ZB):
            zb[i, pl.ds(0, 16)] = jnp.zeros((16,), f32)
            zb[i, pl.ds(16, 16)] = jnp.zeros((16,), f32)
        for q in range(ZB // 16):
            zv[pl.ds(q * 16, 16)] = jnp.zeros((16,), f32)

        sbase = ss * STRIPE

        def one_pass(hd, win0, nwin_sc, orow):
            def zcp(b, _):
                pltpu.sync_copy(zb, nums.at[pl.ds(sbase + b * ZB, ZB)])
                pltpu.sync_copy(zv, dens.at[pl.ds(sbase + b * ZB, ZB)])
                return 0

            lax.fori_loop(0, STRIPE // ZB, zcp, 0)
            plsc.subcore_barrier()

            nwin_my = (nwin_sc - ss + 15) // 16

            def window(j, _):
                base = pl.multiple_of((win0 + j * 16 + ss) * W2, WIN)
                d1 = pltpu.async_copy(src_hbm.at[pl.ds(base, W2)], srcv, sem)
                dd = [pltpu.async_copy(
                    dst_hbm.at[pl.ds(pl.multiple_of(base + b * WIN, WIN), WIN)],
                    dstvs[b], sem_ds[b]) for b in range(WB)]
                d3 = pltpu.async_copy(
                    w_hbm.at[pl.ds(pl.multiple_of(hd * E + base, WIN), W2)],
                    wv.at[pl.ds(16, W2)], sem_c)
                d1.wait()
                off = hd * NP
                for g in range(W2 // 16):
                    sl16 = pl.ds((g % 8) * 16, 16)
                    gidxs[g // 8][sl16] = srcv[pl.ds(g * 16, 16)] + off
                dg = [pltpu.async_copy(h_hbm.at[gidxs[b]],
                                       rows.at[pl.ds(b * WIN, WIN)],
                                       sem_gs[b]) for b in range(WB)]
                d3.wait()
                for b in range(WB):
                    dg[b].wait()

                for i in range(W2):
                    wb = plsc.load_gather(wv, [_full16(16 + i)])
                    rows[i, pl.ds(0, 16)] = rows[i, pl.ds(0, 16)] * wb
                    rows[i, pl.ds(16, 16)] = rows[i, pl.ds(16, 16)] * wb
                for b in range(WB):
                    dd[b].wait()
                    pltpu.sync_copy(rows.at[pl.ds(b * WIN, WIN)],
                                    nums.at[dstvs[b]], add=True)
                    pltpu.sync_copy(wv.at[pl.ds(16 + b * WIN, WIN)],
                                    dens.at[dstvs[b]], add=True)
                return 0

            lax.fori_loop(0, nwin_my, window, 0)
            plsc.subcore_barrier()

            pltpu.sync_copy(
                nums.at[pl.ds(sbase, STRIPE)],
                num_hbm.at[pl.ds(pl.multiple_of(orow * NP + sbase, ZB), STRIPE)])
            pltpu.sync_copy(
                dens.at[pl.ds(sbase, STRIPE)],
                den_hbm.at[pl.ds(pl.multiple_of(orow * NP + sbase, ZB), STRIPE)])
            plsc.subcore_barrier()

        if layer == 1:
            def kbody(k, _):
                one_pass(cc * 2 + k, 0, NW2, cc * 2 + k)
                return 0

            lax.fori_loop(0, n_pass, kbody, 0)
        else:
            one_pass(jnp.zeros((), i32), cc * (NW2 // 2), NW2 // 2, cc)

    return p2


_p2_l1 = _make_p2(1)
_p2_l2 = _make_p2(2)


def _fold(W, av, heads, c):
    return (W.reshape(-1, heads, c) * av[0][None]).sum(-1)


def kernel(node_features, edge_index, edge_attr, W1, as1, ad1, We1, ae1, b1,
           W2, as2, ad2, We2, ae2, b2, gamma, beta, Wp, bp, Wd, bd):
    src = edge_index[0].astype(i32)
    dst = edge_index[1].astype(i32)
    xp = jnp.pad(node_features, ((0, NP - N), (0, 0)))

    Was1 = _fold(W1, as1, H, C)
    Wad1 = _fold(W1, ad1, H, C)
    z16 = jnp.zeros((16,), f32)
    wea1 = jnp.concatenate(
        [z16, jnp.pad(_fold(We1, ae1, H, C), ((0, 0), (0, 4 - H))).reshape(16)])
    Wsd2 = jnp.concatenate(
        [_fold(W2, as2, 1, C), _fold(W2, ad2, 1, C)], axis=1)
    wea2 = jnp.concatenate(
        [z16, jnp.pad(_fold(We2, ae2, 1, C), ((0, 0), (0, 3))).reshape(16)])
    ea_flat = edge_attr.reshape(E * ED)

    W1h = W1.reshape(D, H, C).transpose(1, 0, 2)
    h1 = _tc_h(xp, W1h, H, C).reshape(H * NP, C)
    as1t, ad1t = _tc_sd(xp, Was1, Wad1)
    w1 = _p1_l1(src, dst, ea_flat, as1t.reshape(-1), ad1t.reshape(-1), wea1)
    num1, den1 = _p2_l1(src, dst, w1, h1)

    b1r = b1.reshape(H, C)
    W2r = W2.reshape(H, C, C)
    Wsd2r = Wsd2.reshape(H, C, 2)
    h2, as2t, ad2t = _tc_mid(num1.reshape(H, NP, C), den1.reshape(H, NP),
                             b1r, W2r, Wsd2r)

    w2 = _p1_l2(src, dst, ea_flat, as2t.reshape(-1), ad2t.reshape(-1), wea2)
    num2, den2 = _p2_l2(src, dst, w2, h2)

    op, od = _tc_fin(num2.reshape(2, NP, C), den2.reshape(2, NP),
                     b2.reshape(1, C), gamma.reshape(1, C), beta.reshape(1, C),
                     Wp, bp.reshape(1, 8), Wd, bd.reshape(1, 4))
    return (op[:N], od[:N])
